# baseline (speedup 1.0000x reference)
import jax
import jax.numpy as jnp
from jax import lax
from jax.experimental import pallas as pl
from jax.experimental.pallas import tpu as pltpu
from jax.experimental.pallas import tpu_sc as plsc

B = 1024
D = 128
OUT = 100000
K = 512
T = 0.07

NC = 2
NS = 16
L = 16
NW = NC * NS
BPW = B // NW
CHUNK = 64
CPB = K // CHUNK
NCHUNK = BPW * CPB
XW = BPW * D
_BITREV = (0, 8, 4, 12, 2, 10, 6, 14, 1, 9, 5, 13, 3, 11, 7, 15)


def _sc_body(x_hbm, y_hbm, idx_hbm, mem_hbm, out_hbm,
             x_v, y_v, idx_v, rows0, rows1, ob,
             sem_g0, sem_g1):
    wid = lax.axis_index("s") * NC + lax.axis_index("c")

    pltpu.sync_copy(x_hbm.at[pl.ds(wid * XW, XW)], x_v)
    pltpu.sync_copy(y_hbm.at[pl.ds(wid * BPW, BPW)], y_v)
    pltpu.sync_copy(idx_hbm.at[pl.ds(wid * NCHUNK, NCHUNK)], idx_v)

    inv_t = jnp.float32(1.0 / T)
    lanes = lax.broadcasted_iota(jnp.int32, (L,), 0)
    flips = {lvl: lanes ^ lvl for lvl in (8, 4, 2, 1)}
    masks = {lvl: (lanes & lvl) == 0 for lvl in (8, 4, 2, 1)}
    splitter = jnp.float32(65537.0)

    def _bf16_round(w):
        c = w * splitter
        return c - (c - w)

    def _scale(i, carry):
        x_v[pl.ds(i * L, L)] = _bf16_round(x_v[pl.ds(i * L, L)]) * inv_t
        return carry
    lax.fori_loop(0, XW // L, _scale, 0)

    for bc in range(BPW // L):
        yv = y_v[pl.ds(bc * L, L)]
        for i in range(L):
            b = bc * L + i
            yb = yv.at[jnp.full((L,), i, jnp.int32)].get(
                mode="promise_in_bounds")
            cur = idx_v[b * CPB, pl.ds(0, L)]
            idx_v[b * CPB, pl.ds(0, L)] = jnp.where(lanes == 0, yb, cur)

    def _issue(t, rows, sem):
        pltpu.async_copy(mem_hbm.at[idx_v.at[t]], rows, sem)

    def _gwait(t, rows, sem):
        pltpu.make_async_copy(mem_hbm.at[idx_v.at[t]], rows, sem).wait()

    def _compute(t, rows):
        xbase = (t >> 2) * D
        xvs = [x_v[pl.ds(xbase + jc * L, L)] for jc in range(D // L)]

        def _group(g, carry):
            base = g * L
            accs = []
            for i in range(L):
                r = base + i
                prods = [_bf16_round(rows[r, pl.ds(jc * L, L)]) * xvs[jc]
                         for jc in range(D // L)]
                while len(prods) > 1:
                    prods = [prods[2 * i] + prods[2 * i + 1]
                             for i in range(len(prods) // 2)]
                accs.append(prods[0])
            accs = [accs[p] for p in _BITREV]
            for lvl in (8, 4, 2, 1):
                flip = flips[lvl]
                m = masks[lvl]
                nxt = []
                for i in range(len(accs) // 2):
                    u, v = accs[2 * i], accs[2 * i + 1]
                    us = u + u.at[flip].get(mode="promise_in_bounds")
                    vs = v + v.at[flip].get(mode="promise_in_bounds")
                    nxt.append(jnp.where(m, us, vs))
                accs = nxt
            ob[pl.ds(t * CHUNK + base, L)] = accs[0]
            return carry

        lax.fori_loop(0, CHUNK // L, _group, 0, unroll=2)

    _issue(0, rows0, sem_g0)

    def _pair(p, carry):
        t0 = 2 * p
        t1 = t0 + 1
        _issue(t1, rows1, sem_g1)
        _gwait(t0, rows0, sem_g0)

        _compute(t0, rows0)

        @pl.when(p < NCHUNK // 2 - 1)
        def _():
            _issue(t0 + 2, rows0, sem_g0)
        _gwait(t1, rows1, sem_g1)

        _compute(t1, rows1)
        return carry

    lax.fori_loop(0, NCHUNK // 2, _pair, 0)

    pltpu.sync_copy(ob, out_hbm.at[pl.ds(wid * NCHUNK * CHUNK, NCHUNK * CHUNK)])


def _norm_body(s_ref, o_ref, z_s):
    i = pl.program_id(0)

    @pl.when(i == 0)
    def _():
        z_s[0] = jnp.sum(jnp.exp(s_ref[...]))

    @pl.when(i == 1)
    def _():
        scale = (jnp.float32(B) * jnp.float32(K)) / (jnp.float32(OUT) * z_s[0])
        o_ref[...] = jnp.exp(s_ref[...]) * scale


@jax.jit
def kernel(x, y, memory, idx):
    x_w = x.reshape(B * D)
    idx_r = idx.reshape(B * CPB, CHUNK)

    mesh = plsc.VectorSubcoreMesh(core_axis_name="c", subcore_axis_name="s")
    sc_fn = pl.kernel(
        _sc_body,
        out_type=jax.ShapeDtypeStruct((B * K,), jnp.float32),
        mesh=mesh,
        scratch_types=[
            pltpu.VMEM((XW,), jnp.float32),
            pltpu.VMEM((BPW,), jnp.int32),
            pltpu.VMEM((NCHUNK, CHUNK), jnp.int32),
            pltpu.VMEM((CHUNK, D), jnp.float32),
            pltpu.VMEM((CHUNK, D), jnp.float32),
            pltpu.VMEM((NCHUNK * CHUNK,), jnp.float32),
            pltpu.SemaphoreType.DMA,
            pltpu.SemaphoreType.DMA,
        ],
    )
    scores = sc_fn(x_w, y, idx_r, memory).reshape(B * CPB, CHUNK)

    out = pl.pallas_call(
        _norm_body,
        grid=(2,),
        out_shape=jax.ShapeDtypeStruct((B * CPB, CHUNK), jnp.float32),
        in_specs=[pl.BlockSpec((B * CPB, CHUNK), lambda i: (0, 0))],
        out_specs=pl.BlockSpec((B * CPB, CHUNK), lambda i: (0, 0)),
        scratch_shapes=[pltpu.SMEM((1,), jnp.float32)],
    )(scores)
    return out.reshape(B, K)


# speedup vs baseline: 1.2843x; 1.2843x over previous
import jax
import jax.numpy as jnp
from jax import lax
from jax.experimental import pallas as pl
from jax.experimental.pallas import tpu as pltpu
from jax.experimental.pallas import tpu_sc as plsc

B = 1024
D = 128
OUT = 100000
K = 512
T = 0.07

NC = 2
NS = 16
L = 16
NW = NC * NS
BPW = B // NW
CHUNK = 128
CPB = K // CHUNK
NCHUNK = BPW * CPB
XW = BPW * D
_BITREV = (0, 8, 4, 12, 2, 10, 6, 14, 1, 9, 5, 13, 3, 11, 7, 15)


def _sc_body(x_hbm, y_hbm, idx_hbm, mem_hbm, out_hbm,
             x_v, y_v, idx_v, rows0, rows1, ob,
             sem_g0, sem_g1):
    wid = lax.axis_index("s") * NC + lax.axis_index("c")

    pltpu.sync_copy(x_hbm.at[pl.ds(wid * XW, XW)], x_v)
    pltpu.sync_copy(y_hbm.at[pl.ds(wid * BPW, BPW)], y_v)
    pltpu.sync_copy(idx_hbm.at[pl.ds(wid * NCHUNK, NCHUNK)], idx_v)

    inv_t = jnp.float32(1.0 / T)
    lanes = lax.broadcasted_iota(jnp.int32, (L,), 0)
    flips = {lvl: lanes ^ lvl for lvl in (8, 4, 2, 1)}
    masks = {lvl: (lanes & lvl) == 0 for lvl in (8, 4, 2, 1)}
    splitter = jnp.float32(65537.0)

    def _bf16_round(w):
        c = w * splitter
        return c - (c - w)

    def _scale(i, carry):
        x_v[pl.ds(i * L, L)] = _bf16_round(x_v[pl.ds(i * L, L)]) * inv_t
        return carry
    lax.fori_loop(0, XW // L, _scale, 0)

    for bc in range(BPW // L):
        yv = y_v[pl.ds(bc * L, L)]
        for i in range(L):
            b = bc * L + i
            yb = yv.at[jnp.full((L,), i, jnp.int32)].get(
                mode="promise_in_bounds")
            cur = idx_v[b * CPB, pl.ds(0, L)]
            idx_v[b * CPB, pl.ds(0, L)] = jnp.where(lanes == 0, yb, cur)

    def _issue(t, rows, sem):
        pltpu.async_copy(mem_hbm.at[idx_v.at[t]], rows, sem)

    def _gwait(t, rows, sem):
        pltpu.make_async_copy(mem_hbm.at[idx_v.at[t]], rows, sem).wait()

    def _compute(t, rows):
        xbase = (t >> 2) * D
        xvs = [x_v[pl.ds(xbase + jc * L, L)] for jc in range(D // L)]

        def _group(g, carry):
            base = g * L
            accs = []
            for i in range(L):
                r = base + i
                prods = [_bf16_round(rows[r, pl.ds(jc * L, L)]) * xvs[jc]
                         for jc in range(D // L)]
                while len(prods) > 1:
                    prods = [prods[2 * i] + prods[2 * i + 1]
                             for i in range(len(prods) // 2)]
                accs.append(prods[0])
            res = jnp.zeros((L,), jnp.float32)
            last = jnp.full((L,), L - 1, jnp.int32)
            for i in range(L):
                cs = jnp.cumsum(accs[i])
                tot = cs.at[last].get(mode="promise_in_bounds")
                res = jnp.where(lanes == i, tot, res)
            ob[pl.ds(t * CHUNK + base, L)] = res
            return carry

        lax.fori_loop(0, CHUNK // L, _group, 0, unroll=2)

    _issue(0, rows0, sem_g0)

    def _pair(p, carry):
        t0 = 2 * p
        t1 = t0 + 1
        _issue(t1, rows1, sem_g1)
        _gwait(t0, rows0, sem_g0)

        _compute(t0, rows0)

        @pl.when(p < NCHUNK // 2 - 1)
        def _():
            _issue(t0 + 2, rows0, sem_g0)
        _gwait(t1, rows1, sem_g1)

        _compute(t1, rows1)
        return carry

    lax.fori_loop(0, NCHUNK // 2, _pair, 0)

    pltpu.sync_copy(ob, out_hbm.at[pl.ds(wid * NCHUNK * CHUNK, NCHUNK * CHUNK)])


def _norm_body(s_ref, o_ref, z_s):
    i = pl.program_id(0)

    @pl.when(i == 0)
    def _():
        z_s[0] = jnp.sum(jnp.exp(s_ref[...]))

    @pl.when(i == 1)
    def _():
        scale = (jnp.float32(B) * jnp.float32(K)) / (jnp.float32(OUT) * z_s[0])
        o_ref[...] = jnp.exp(s_ref[...]) * scale


@jax.jit
def kernel(x, y, memory, idx):
    x_w = x.reshape(B * D)
    idx_r = idx.reshape(B * CPB, CHUNK)

    mesh = plsc.VectorSubcoreMesh(core_axis_name="c", subcore_axis_name="s")
    sc_fn = pl.kernel(
        _sc_body,
        out_type=jax.ShapeDtypeStruct((B * K,), jnp.float32),
        mesh=mesh,
        compiler_params=pltpu.CompilerParams(needs_layout_passes=False),
        scratch_types=[
            pltpu.VMEM((XW,), jnp.float32),
            pltpu.VMEM((BPW,), jnp.int32),
            pltpu.VMEM((NCHUNK, CHUNK), jnp.int32),
            pltpu.VMEM((CHUNK, D), jnp.float32),
            pltpu.VMEM((CHUNK, D), jnp.float32),
            pltpu.VMEM((NCHUNK * CHUNK,), jnp.float32),
            pltpu.SemaphoreType.DMA,
            pltpu.SemaphoreType.DMA,
        ],
    )
    scores = sc_fn(x_w, y, idx_r, memory).reshape(B * CPB, CHUNK)

    out = pl.pallas_call(
        _norm_body,
        grid=(2,),
        out_shape=jax.ShapeDtypeStruct((B * CPB, CHUNK), jnp.float32),
        in_specs=[pl.BlockSpec((B * CPB, CHUNK), lambda i: (0, 0))],
        out_specs=pl.BlockSpec((B * CPB, CHUNK), lambda i: (0, 0)),
        scratch_shapes=[pltpu.SMEM((1,), jnp.float32)],
    )(scores)
    return out.reshape(B, K)
